# baseline (speedup 1.0000x reference)
import functools

import jax
import jax.numpy as jnp
from jax import lax
from jax.experimental import pallas as pl
from jax.experimental.pallas import tpu as pltpu
from jax.experimental.pallas import tpu_sc as plsc

_CHUNK = 128
_K = 5
_N_WORKERS = 32


@functools.partial(jax.jit, static_argnums=(2,))
def _sc_gather(table, tok2d, iters):
    n_rows, _ = tok2d.shape
    t_total = n_rows * _CHUNK
    d = table.shape[1]
    per_w_rows = n_rows // _N_WORKERS
    step = _K * _CHUNK

    mesh = plsc.VectorSubcoreMesh(core_axis_name="c", subcore_axis_name="s")

    @functools.partial(
        pl.kernel,
        mesh=mesh,
        out_type=jax.ShapeDtypeStruct((t_total, d), jnp.float32),
        scratch_types=[
            pltpu.VMEM((2, _K, _CHUNK), jnp.int32),
            pltpu.VMEM((2, step, d), jnp.float32),
            pltpu.SemaphoreType.DMA,
            pltpu.SemaphoreType.DMA,
            pltpu.SemaphoreType.DMA,
            pltpu.SemaphoreType.DMA,
            pltpu.SemaphoreType.DMA,
            pltpu.SemaphoreType.DMA,
        ],
        compiler_params=pltpu.CompilerParams(use_tc_tiling_on_sc=False),
    )
    def k(table_hbm, tok_hbm, out_hbm, idx_v, rows_v, g0, g1, s0, s1, i0, i1):
        gsem = (g0, g1)
        ssem = (s0, s1)
        isem = (i0, i1)
        n_cores = lax.axis_size("c")
        wid = lax.axis_index("s") * n_cores + lax.axis_index("c")
        row_base = wid * per_w_rows

        def wait_rows(sem):
            pltpu.make_async_copy(
                out_hbm.at[pl.ds(0, step)], rows_v.at[0], sem
            ).wait()

        def wait_idx(sem):
            pltpu.make_async_copy(
                tok_hbm.at[pl.ds(0, _K)], idx_v.at[0], sem
            ).wait()

        def start_idx(gidx, b):
            pltpu.make_async_copy(
                tok_hbm.at[pl.ds(row_base + gidx * _K, _K)],
                idx_v.at[b],
                isem[b],
            ).start()

        def fire_gathers(b):
            for j in range(_K):
                pltpu.make_async_copy(
                    table_hbm.at[idx_v.at[b, j]],
                    rows_v.at[b, pl.ds(j * _CHUNK, _CHUNK)],
                    gsem[b],
                ).start()

        def start_store(gidx, b):
            pltpu.make_async_copy(
                rows_v.at[b],
                out_hbm.at[pl.ds((row_base + gidx * _K) * _CHUNK, step)],
                ssem[b],
            ).start()

        start_idx(0, 0)

        def pair_body(p, carry):
            for b in (0, 1):
                g = 2 * p + b
                ob = 1 - b
                if b == 1:
                    wait_rows(gsem[ob])
                    start_store(g - 1, ob)
                else:

                    @pl.when(p >= 1)
                    def _():
                        wait_rows(gsem[ob])
                        start_store(g - 1, ob)

                @pl.when(p >= 1)
                def _():
                    wait_rows(ssem[b])

                wait_idx(isem[b])
                fire_gathers(b)
                start_idx(jnp.minimum(g + 1, iters - 1), ob)
            return carry

        lax.fori_loop(0, iters // 2, pair_body, 0)

        wait_rows(gsem[1])
        start_store(iters - 1, 1)
        wait_idx(isem[0])
        wait_rows(ssem[0])
        wait_rows(ssem[1])

    return k(table, tok2d)


def _mm_body(g_ref, w_ref, b_ref, o_ref):
    bmh = g_ref.shape[0]
    n_cb = g_ref.shape[1]
    acc = None
    for cb in range(n_cb):
        blk = g_ref[:, cb, :, :].reshape(bmh * 8, 128)
        p = jnp.dot(blk, w_ref[cb], preferred_element_type=jnp.float32)
        acc = p if acc is None else acc + p
    o_ref[...] = jnp.maximum(acc + b_ref[...], 0.0)


@jax.jit
def _tc_matmul(g4, w3, fc_b2d):
    n_bh, n_cb = g4.shape[0], g4.shape[1]
    bm = 256
    bmh = bm // 8
    return pl.pallas_call(
        _mm_body,
        grid=(n_bh // bmh,),
        in_specs=[
            pl.BlockSpec((bmh, n_cb, 8, 128), lambda i: (i, 0, 0, 0)),
            pl.BlockSpec((n_cb, 128, 64), lambda i: (0, 0, 0)),
            pl.BlockSpec((1, 64), lambda i: (0, 0)),
        ],
        out_specs=pl.BlockSpec((bm, 64), lambda i: (i, 0)),
        out_shape=jax.ShapeDtypeStruct((n_bh * 8, 64), jnp.float32),
    )(g4, w3, fc_b2d)


def kernel(tokens, embed_table, fc_w, fc_b):
    batch, seq = tokens.shape
    d = embed_table.shape[1]
    t_total = batch * seq
    n_cb = seq // 2
    n_rb = batch // 8
    iters = t_total // (_N_WORKERS * _K * _CHUNK)

    tok_perm = (
        tokens.astype(jnp.int32)
        .reshape(n_rb, 8, n_cb, 2)
        .transpose(0, 2, 1, 3)
        .reshape(t_total // _CHUNK, _CHUNK)
    )

    gathered = _sc_gather(embed_table, tok_perm, iters)
    g4 = gathered.reshape(batch // 8, n_cb, 8, 2 * d)
    w3 = fc_w.reshape(d, n_cb, 2 * d).transpose(1, 2, 0)
    return _tc_matmul(g4, w3, fc_b.reshape(1, d))


# speedup vs baseline: 1.2967x; 1.2967x over previous
import functools

import jax
import jax.numpy as jnp
from jax import lax
from jax.experimental import pallas as pl
from jax.experimental.pallas import tpu as pltpu
from jax.experimental.pallas import tpu_sc as plsc

_CHUNK = 128
_K = 5
_N_WORKERS = 32


@functools.partial(jax.jit, static_argnums=(2,))
def _sc_gather(table, tok2d, iters):
    n_rows, _ = tok2d.shape
    t_total = n_rows * _CHUNK
    d = table.shape[1]
    per_w_rows = n_rows // _N_WORKERS
    step = _K * _CHUNK

    mesh = plsc.VectorSubcoreMesh(core_axis_name="c", subcore_axis_name="s")

    @functools.partial(
        pl.kernel,
        mesh=mesh,
        out_type=jax.ShapeDtypeStruct((t_total, d), jnp.float32),
        scratch_types=[
            pltpu.VMEM((2, _K, _CHUNK), jnp.int32),
            pltpu.VMEM((2, step, d), jnp.float32),
            pltpu.SemaphoreType.DMA,
            pltpu.SemaphoreType.DMA,
            pltpu.SemaphoreType.DMA,
            pltpu.SemaphoreType.DMA,
            pltpu.SemaphoreType.DMA,
            pltpu.SemaphoreType.DMA,
        ],
        compiler_params=pltpu.CompilerParams(use_tc_tiling_on_sc=False),
    )
    def k(table_hbm, tok_hbm, out_hbm, idx_v, rows_v, g0, g1, s0, s1, i0, i1):
        gsem = (g0, g1)
        ssem = (s0, s1)
        isem = (i0, i1)
        n_cores = lax.axis_size("c")
        wid = lax.axis_index("s") * n_cores + lax.axis_index("c")
        row_base = wid * per_w_rows

        def wait_rows(sem):
            pltpu.make_async_copy(
                out_hbm.at[pl.ds(0, step)], rows_v.at[0], sem
            ).wait()

        def wait_idx(sem):
            pltpu.make_async_copy(
                tok_hbm.at[pl.ds(0, _K)], idx_v.at[0], sem
            ).wait()

        def start_idx(gidx, b):
            pltpu.make_async_copy(
                tok_hbm.at[pl.ds(row_base + gidx * _K, _K)],
                idx_v.at[b],
                isem[b],
            ).start()

        def fire_gathers(b):
            for j in range(_K):
                pltpu.make_async_copy(
                    table_hbm.at[idx_v.at[b, j]],
                    rows_v.at[b, pl.ds(j * _CHUNK, _CHUNK)],
                    gsem[b],
                ).start()

        def start_store(gidx, b):
            pltpu.make_async_copy(
                rows_v.at[b],
                out_hbm.at[pl.ds((row_base + gidx * _K) * _CHUNK, step)],
                ssem[b],
            ).start()

        start_idx(0, 0)

        def pair_body(p, carry):
            for b in (0, 1):
                g = 2 * p + b
                ob = 1 - b
                if b == 1:
                    wait_rows(gsem[ob])
                    start_store(g - 1, ob)
                else:

                    @pl.when(p >= 1)
                    def _():
                        wait_rows(gsem[ob])
                        start_store(g - 1, ob)

                @pl.when(p >= 1)
                def _():
                    wait_rows(ssem[b])

                wait_idx(isem[b])
                fire_gathers(b)
                start_idx(jnp.minimum(g + 1, iters - 1), ob)
            return carry

        lax.fori_loop(0, iters // 2, pair_body, 0)

        wait_rows(gsem[1])
        start_store(iters - 1, 1)
        wait_idx(isem[0])
        wait_rows(ssem[0])
        wait_rows(ssem[1])

    return k(table, tok2d)


def _mm_body(g_ref, w_ref, b_ref, o_ref):
    seq = g_ref.shape[0]
    acc = None
    for l in range(seq):
        p = jnp.dot(g_ref[l], w_ref[l], preferred_element_type=jnp.float32)
        acc = p if acc is None else acc + p
    o_ref[...] = jnp.maximum(acc + b_ref[...], 0.0)


@jax.jit
def _tc_matmul(g3, w2, fc_b2d):
    seq, n_pair = g3.shape[0], g3.shape[1]
    mb = 128
    return pl.pallas_call(
        _mm_body,
        grid=(n_pair // mb,),
        in_specs=[
            pl.BlockSpec((seq, mb, 128), lambda i: (0, i, 0)),
            pl.BlockSpec((seq, 128, 128), lambda i: (0, 0, 0)),
            pl.BlockSpec((1, 128), lambda i: (0, 0)),
        ],
        out_specs=pl.BlockSpec((mb, 128), lambda i: (i, 0)),
        out_shape=jax.ShapeDtypeStruct((n_pair, 128), jnp.float32),
    )(g3, w2, fc_b2d)


def kernel(tokens, embed_table, fc_w, fc_b):
    batch, seq = tokens.shape
    d = embed_table.shape[1]
    t_total = batch * seq
    iters = t_total // (_N_WORKERS * _K * _CHUNK)

    tok2d = tokens.astype(jnp.int32).T.reshape(t_total // _CHUNK, _CHUNK)

    gathered = _sc_gather(embed_table, tok2d, iters)
    g3 = gathered.reshape(seq, batch // 2, 2 * d)

    wlt = fc_w.reshape(d, seq, d).transpose(1, 2, 0)
    w2 = jnp.zeros((seq, 2 * d, 2 * d), jnp.float32)
    w2 = w2.at[:, :d, :d].set(wlt).at[:, d:, d:].set(wlt)
    b2 = jnp.concatenate([fc_b, fc_b]).reshape(1, 2 * d)

    out_pairs = _tc_matmul(g3, w2, b2)
    return out_pairs.reshape(batch, d)
